# baseline (speedup 1.0000x reference)
import functools

import jax
import jax.numpy as jnp
from jax.experimental import pallas as pl

_INPUT_SIZE = 4
_N_CLASSES = 16
_HIDDEN = 100
_PAD_VALUE = -100.0
_BT = 8


def _mlp_kernel(x_ref, w1_ref, w2_ref, o_ref, *, t_tile):
    j = pl.program_id(1)
    xs = x_ref[:, pl.ds(j * t_tile, t_tile + _INPUT_SIZE - 1)]
    y = jnp.concatenate(
        [xs[:, k:k + t_tile] for k in range(_INPUT_SIZE)]
        + [jnp.ones((1, t_tile), jnp.float32)], axis=0)
    h = jnp.dot(w1_ref[:, :], y, preferred_element_type=jnp.float32)
    h = jnp.maximum(h, 0.0)
    h = jnp.concatenate([h, jnp.ones((1, t_tile), jnp.float32)], axis=0)
    logits = jnp.dot(w2_ref[:, :], h, preferred_element_type=jnp.float32)
    lg = logits.reshape(_BT, _N_CLASSES, t_tile)
    m = jnp.max(lg, axis=1, keepdims=True)
    shifted = lg - m
    lse = jnp.log(jnp.sum(jnp.exp(shifted), axis=1, keepdims=True))
    res = (shifted - lse).reshape(_BT * _N_CLASSES, t_tile)
    zt = jnp.transpose(res)
    for b in range(_BT):
        o_ref[b, :, :] = zt[:, b * _N_CLASSES:(b + 1) * _N_CLASSES]


@jax.jit
def kernel(input_, W1, b1, W2, b2):
    B, T = input_.shape
    TT = 2048
    left = jnp.full((B, _INPUT_SIZE - 1), _PAD_VALUE, input_.dtype)
    right = jnp.zeros((B, 128 - (_INPUT_SIZE - 1)), input_.dtype)
    xp = jnp.concatenate([left, input_, right], axis=1)

    bi = jnp.arange(_BT)
    w1b = jnp.zeros((_BT * _HIDDEN, _INPUT_SIZE * _BT + 1), jnp.float32)
    w1b = w1b.at[
        (bi[:, None, None] * _HIDDEN + jnp.arange(_HIDDEN)[None, :, None]),
        (jnp.arange(_INPUT_SIZE)[None, None, :] * _BT + bi[:, None, None]),
    ].set(jnp.transpose(W1)[None, :, :])
    w1b = w1b.at[:, -1].set(jnp.tile(b1, _BT))
    w2b = jnp.zeros((_BT * _N_CLASSES, _BT * _HIDDEN + 1), jnp.float32)
    w2b = w2b.at[
        (bi[:, None, None] * _N_CLASSES
         + jnp.arange(_N_CLASSES)[None, None, :]),
        (bi[:, None, None] * _HIDDEN + jnp.arange(_HIDDEN)[None, :, None]),
    ].set(jnp.transpose(W2)[None, :, :].transpose(0, 2, 1))
    w2b = w2b.at[:, -1].set(jnp.tile(b2, _BT))

    out = pl.pallas_call(
        functools.partial(_mlp_kernel, t_tile=TT),
        grid=(B // _BT, T // TT),
        in_specs=[
            pl.BlockSpec((_BT, xp.shape[1]), lambda i, j: (i, 0)),
            pl.BlockSpec(w1b.shape, lambda i, j: (0, 0)),
            pl.BlockSpec(w2b.shape, lambda i, j: (0, 0)),
        ],
        out_specs=pl.BlockSpec((_BT, TT, _N_CLASSES), lambda i, j: (i, j, 0)),
        out_shape=jax.ShapeDtypeStruct((B, T, _N_CLASSES), jnp.float32),
    )(xp, w1b, w2b)
    return out


# speedup vs baseline: 1.4891x; 1.4891x over previous
import jax
import jax.numpy as jnp
from jax.experimental import pallas as pl

_INPUT_SIZE = 4
_N_CLASSES = 16
_HIDDEN = 100
_PAD_VALUE = -100.0
_PH = 8


def _mlp_kernel(x_ref, w1_ref, w2_ref, o_ref):
    nr = o_ref.shape[1]
    xt = jnp.transpose(x_ref[0])
    x11 = jnp.concatenate(
        [xt[:, 0:nr], xt[0:_INPUT_SIZE - 1, 1:nr + 1],
         jnp.ones((1, nr), jnp.float32)], axis=0)
    h = jnp.dot(w1_ref[:, :], x11, preferred_element_type=jnp.float32)
    h = jnp.maximum(h, 0.0)
    h = jnp.concatenate([h, jnp.ones((1, nr), jnp.float32)], axis=0)
    logits = jnp.dot(w2_ref[:, :], h, preferred_element_type=jnp.float32)
    lg = logits.reshape(_PH, _N_CLASSES, nr)
    m = jnp.max(lg, axis=1, keepdims=True)
    shifted = lg - m
    lse = jnp.log(jnp.sum(jnp.exp(shifted), axis=1, keepdims=True))
    res = (shifted - lse).reshape(_PH * _N_CLASSES, nr)
    o_ref[0, :, :] = jnp.transpose(res)


@jax.jit
def kernel(input_, W1, b1, W2, b2):
    B, T = input_.shape
    left = jnp.full((B, _INPUT_SIZE - 1), _PAD_VALUE, input_.dtype)
    right = jnp.zeros((B, 128 - (_INPUT_SIZE - 1)), input_.dtype)
    xp = jnp.concatenate([left, input_, right], axis=1)
    xp3 = xp.reshape(B, xp.shape[1] // _PH, _PH)

    ii = jnp.arange(_PH)
    jj = jnp.arange(_HIDDEN)
    kk = jnp.arange(_INPUT_SIZE)
    cc = jnp.arange(_N_CLASSES)
    w1s = jnp.zeros((_PH * _HIDDEN, _PH + _INPUT_SIZE), jnp.float32)
    w1s = w1s.at[
        ii[:, None, None] * _HIDDEN + jj[None, :, None],
        ii[:, None, None] + kk[None, None, :],
    ].set(jnp.transpose(W1)[None, :, :])
    w1s = w1s.at[:, -1].set(jnp.tile(b1, _PH))
    w2s = jnp.zeros((_PH * _N_CLASSES, _PH * _HIDDEN + 1), jnp.float32)
    w2s = w2s.at[
        ii[:, None, None] * _N_CLASSES + cc[None, None, :],
        ii[:, None, None] * _HIDDEN + jj[None, :, None],
    ].set(W2[None, :, :])
    w2s = w2s.at[:, -1].set(jnp.tile(b2, _PH))

    out = pl.pallas_call(
        _mlp_kernel,
        grid=(B,),
        in_specs=[
            pl.BlockSpec((1,) + xp3.shape[1:], lambda i: (i, 0, 0)),
            pl.BlockSpec(w1s.shape, lambda i: (0, 0)),
            pl.BlockSpec(w2s.shape, lambda i: (0, 0)),
        ],
        out_specs=pl.BlockSpec((1, T // _PH, _PH * _N_CLASSES),
                               lambda i: (i, 0, 0)),
        out_shape=jax.ShapeDtypeStruct((B, T // _PH, _PH * _N_CLASSES),
                                       jnp.float32),
    )(xp3, w1s, w2s)
    return out.reshape(B, T, _N_CLASSES)
